# baseline (speedup 1.0000x reference)
import functools

import jax
import jax.numpy as jnp
from jax import lax
from jax.experimental import pallas as pl
from jax.experimental.pallas import tpu as pltpu
from jax.experimental.pallas import tpu_sc as plsc

BATCH = 16384
EMB_DIM = 64
LANES = 16
NUM_CORES = 2
NUM_SUBCORES = 16
NUM_WORKERS = NUM_CORES * NUM_SUBCORES
ROWS_PER_WORKER = BATCH // NUM_WORKERS
CHUNK = 128
NUM_CHUNKS = ROWS_PER_WORKER // CHUNK

_mesh = plsc.VectorSubcoreMesh(core_axis_name="c", subcore_axis_name="s")


@functools.partial(
    pl.kernel,
    mesh=_mesh,
    compiler_params=pltpu.CompilerParams(
        needs_layout_passes=False, use_tc_tiling_on_sc=False),
    out_type=jax.ShapeDtypeStruct((BATCH,), jnp.float32),
    scratch_types=[
        pltpu.VMEM((3 * NUM_CHUNKS, CHUNK), jnp.int32),
        pltpu.VMEM((ROWS_PER_WORKER, EMB_DIM), jnp.float32),
        pltpu.VMEM((ROWS_PER_WORKER, EMB_DIM), jnp.float32),
        pltpu.VMEM((ROWS_PER_WORKER, EMB_DIM), jnp.float32),
        pltpu.VMEM((ROWS_PER_WORKER * LANES,), jnp.float32),
        pltpu.VMEM((ROWS_PER_WORKER,), jnp.float32),
        pltpu.SemaphoreType.DMA,
        pltpu.SemaphoreType.DMA,
    ],
)
def _distmult_sc(hs_hbm, rs_hbm, ts_hbm, ent_hbm, rel_hbm, out_hbm,
                 idx_v, eh_v, er_v, et_v, p_v, o_v, sem_idx, sem_rows):
    wid = lax.axis_index("s") * NUM_CORES + lax.axis_index("c")
    base = wid * ROWS_PER_WORKER

    idx_copies = []
    for j in range(NUM_CHUNKS):
        src = pl.ds(base + j * CHUNK, CHUNK)
        idx_copies.append(
            pltpu.async_copy(hs_hbm.at[src], idx_v.at[j], sem_idx))
        idx_copies.append(
            pltpu.async_copy(rs_hbm.at[src], idx_v.at[NUM_CHUNKS + j], sem_idx))
        idx_copies.append(
            pltpu.async_copy(ts_hbm.at[src], idx_v.at[2 * NUM_CHUNKS + j], sem_idx))
    for c in idx_copies:
        c.wait()

    row_copies = []
    for j in range(NUM_CHUNKS):
        dst = pl.ds(j * CHUNK, CHUNK)
        row_copies.append(
            pltpu.async_copy(ent_hbm.at[idx_v.at[j]], eh_v.at[dst], sem_rows))
        row_copies.append(
            pltpu.async_copy(rel_hbm.at[idx_v.at[NUM_CHUNKS + j]], er_v.at[dst], sem_rows))
        row_copies.append(
            pltpu.async_copy(ent_hbm.at[idx_v.at[2 * NUM_CHUNKS + j]], et_v.at[dst], sem_rows))
    for c in row_copies:
        c.wait()

    def row_body(r, carry):
        acc = (eh_v[r, pl.ds(0, LANES)] * er_v[r, pl.ds(0, LANES)]
               ) * et_v[r, pl.ds(0, LANES)]
        for k in range(1, EMB_DIM // LANES):
            s = pl.ds(k * LANES, LANES)
            acc = acc + (eh_v[r, s] * er_v[r, s]) * et_v[r, s]
        p_v[pl.ds(r * LANES, LANES)] = acc
        return carry

    lax.fori_loop(0, ROWS_PER_WORKER, row_body, 0)

    lane_iota = jax.lax.iota(jnp.int32, LANES)

    def blk_body(b, carry):
        flat = (b * LANES + lane_iota) * LANES
        res = plsc.load_gather(p_v, [flat])
        for i in range(1, LANES):
            res = res + plsc.load_gather(p_v, [flat + i])
        o_v[pl.ds(b * LANES, LANES)] = res
        return carry

    lax.fori_loop(0, ROWS_PER_WORKER // LANES, blk_body, 0)

    pltpu.sync_copy(o_v, out_hbm.at[pl.ds(base, ROWS_PER_WORKER)])


def kernel(hs, rs, ts, ent_embs, rel_embs):
    return _distmult_sc(hs, rs, ts, ent_embs, rel_embs)
